# baseline (speedup 1.0000x reference)
import functools

import jax
import jax.numpy as jnp
from jax.experimental import pallas as pl


def _conv(x, w, b, stride, pad, dt=None):
    if dt is not None:
        x, w = x.astype(dt), w.astype(dt)
    y = jax.lax.conv_general_dilated(x, w, window_strides=(stride, stride),
                                     padding=((pad, pad), (pad, pad)),
                                     dimension_numbers=('NCHW', 'OIHW', 'NCHW'),
                                     preferred_element_type=jnp.float32)
    return y + b[None, :, None, None]


def _convT_dilated(x, w, b):
    y = jax.lax.conv_transpose(x, w, strides=(2, 2),
                               padding=((1, 1), (1, 1)),
                               dimension_numbers=('NCHW', 'OIHW', 'NCHW'),
                               preferred_element_type=jnp.float32)
    return y + b[None, :, None, None]


def _convT(x, w, b):
    O, I, _, _ = w.shape
    W = jnp.stack([w[:, :, (1 - di)::2, :][:, :, :, (1 - dj)::2]
                   for di in (0, 1) for dj in (0, 1)], axis=0)
    W = W.reshape(4 * O, I, 2, 2)
    p = jax.lax.conv_general_dilated(x, W, window_strides=(1, 1),
                                     padding='VALID',
                                     dimension_numbers=('NCHW', 'OIHW', 'NCHW'),
                                     preferred_element_type=jnp.float32)
    B, _, h, wd = p.shape
    p = p.reshape(B, 2, 2, O, h, wd)
    y = p.transpose(0, 3, 4, 1, 5, 2).reshape(B, O, 2 * h, 2 * wd)
    return y + b[None, :, None, None]


_ROWS = 512


def _vq_kernel(z_ref, cb_ref, quant_ref, idx_ref, loss_ref):
    z = z_ref[...]
    cb = cb_ref[...]
    zz = jnp.sum(z * z, axis=1, keepdims=True)
    cc = jnp.sum(cb * cb, axis=1)[None, :]
    cross = jax.lax.dot_general(
        z, cb, (((1,), (1,)), ((), ())),
        preferred_element_type=jnp.float32)
    d2 = zz + cc - 2.0 * cross
    idx = jnp.argmin(d2, axis=1).astype(jnp.int32)
    k = d2.shape[1]
    onehot = (idx[:, None] == jax.lax.broadcasted_iota(jnp.int32, (1, k), 1)
              ).astype(jnp.float32)
    quant = jax.lax.dot_general(
        onehot, cb, (((1,), (0,)), ((), ())),
        preferred_element_type=jnp.float32)
    quant_ref[...] = quant
    idx_ref[...] = idx.reshape(1, 1, -1)
    diff = quant - z
    part = jnp.sum(diff * diff).reshape(1, 1)
    @pl.when(pl.program_id(0) == 0)
    def _():
        loss_ref[...] = jnp.zeros((1, 1), jnp.float32)
    loss_ref[...] += part


def _vq(z_flat, codebook):
    n, d = z_flat.shape
    k = codebook.shape[0]
    nblk = n // _ROWS
    quant, idx, losssum = pl.pallas_call(
        _vq_kernel,
        grid=(nblk,),
        in_specs=[
            pl.BlockSpec((_ROWS, d), lambda i: (i, 0)),
            pl.BlockSpec((k, d), lambda i: (0, 0)),
        ],
        out_specs=[
            pl.BlockSpec((_ROWS, d), lambda i: (i, 0)),
            pl.BlockSpec((1, 1, _ROWS), lambda i: (i, 0, 0)),
            pl.BlockSpec((1, 1), lambda i: (0, 0)),
        ],
        out_shape=[
            jax.ShapeDtypeStruct((n, d), jnp.float32),
            jax.ShapeDtypeStruct((nblk, 1, _ROWS), jnp.int32),
            jax.ShapeDtypeStruct((1, 1), jnp.float32),
        ],
    )(z_flat, codebook)
    return quant, idx.reshape(n), losssum[0, 0]


_T3 = 10


def _convT3_body(xt_ref, w_ref, out_ref):
    for tau in range(110 // _T3):
        acc = jnp.zeros((_T3 * 110, 12), jnp.float32)
        for u in (0, 1):
            base = tau * (_T3 * 110) + u * 110
            lhs = jnp.concatenate(
                [xt_ref[0, base:base + _T3 * 110, :],
                 xt_ref[0, base + 1:base + 1 + _T3 * 110, :]],
                axis=1)
            acc += jax.lax.dot_general(
                lhs, w_ref[u], (((1,), (0,)), ((), ())),
                preferred_element_type=jnp.float32)
        out_ref[0, tau] = acc


def _convT3_pallas(y, w, b):
    B, C, R, S = y.shape
    yt = jnp.transpose(y, (0, 2, 3, 1)).astype(jnp.bfloat16)
    yt = jnp.pad(yt, ((0, 0), (0, 2), (0, 0), (0, 0)))
    yt = yt.reshape(B, 112 * 110, 128)
    wk = jnp.stack([
        jnp.concatenate([
            jnp.stack([w[c, :, 2 * u + 1 - di, 2 * v + 1 - dj]
                       for di in (0, 1) for dj in (0, 1) for c in (0, 1, 2)],
                      axis=1)
            for v in (0, 1)], axis=0)
        for u in (0, 1)], axis=0).astype(jnp.bfloat16)
    ntau = R // _T3
    p = pl.pallas_call(
        _convT3_body,
        grid=(B,),
        in_specs=[
            pl.BlockSpec((1, 112 * 110, 128), lambda bi: (bi, 0, 0)),
            pl.BlockSpec((2, 256, 12), lambda bi: (0, 0, 0)),
        ],
        out_specs=pl.BlockSpec((1, ntau, _T3 * 110, 12),
                               lambda bi: (bi, 0, 0, 0)),
        out_shape=jax.ShapeDtypeStruct((B, ntau, _T3 * 110, 12), jnp.float32),
    )(yt, wk)
    p = p.reshape(B, R, S, 2, 2, 3)
    xr = p.transpose(0, 5, 1, 3, 2, 4).reshape(B, 3, 2 * R, 2 * S)
    xr = xr[:, :, :2 * R - 2, :2 * S - 2]
    return xr + b[None, :, None, None]


def kernel(x, enc_w1, enc_b1, enc_w2, enc_b2, enc_w3, enc_b3, codebook,
           dec_w1, dec_b1, dec_w2, dec_b2, dec_w3, dec_b3):
    beta = 0.25
    h = jax.nn.relu(_conv(x, enc_w1, enc_b1, 2, 1))
    h = jax.nn.relu(_conv(h, enc_w2, enc_b2, 2, 1))
    z = _conv(h, enc_w3, enc_b3, 1, 1)
    B, D, Hh, Ww = z.shape
    z_flat = jnp.transpose(z, (0, 2, 3, 1)).reshape(-1, D)
    quant_flat, indices, losssum = _vq(z_flat, codebook)
    quantized = jnp.transpose(quant_flat.reshape(B, Hh, Ww, D), (0, 3, 1, 2))
    codebook_loss = losssum / jnp.float32(z_flat.size)
    commitment_loss = beta * codebook_loss
    g = jax.nn.relu(_conv(quantized, dec_w1, dec_b1, 1, 1))
    g = jax.nn.relu(_convT_dilated(g, dec_w2, dec_b2))
    x_recon = _convT3_pallas(g, dec_w3, dec_b3)
    return (x_recon, codebook_loss, commitment_loss,
            indices.reshape(B, Hh, Ww))


# speedup vs baseline: 1.3033x; 1.3033x over previous
import functools

import jax
import jax.numpy as jnp
from jax.experimental import pallas as pl


def _conv(x, w, b, stride, pad, dt=None):
    if dt is not None:
        x, w = x.astype(dt), w.astype(dt)
    y = jax.lax.conv_general_dilated(x, w, window_strides=(stride, stride),
                                     padding=((pad, pad), (pad, pad)),
                                     dimension_numbers=('NCHW', 'OIHW', 'NCHW'),
                                     preferred_element_type=jnp.float32)
    return y + b[None, :, None, None]


def _convT_dilated(x, w, b):
    y = jax.lax.conv_transpose(x, w, strides=(2, 2),
                               padding=((1, 1), (1, 1)),
                               dimension_numbers=('NCHW', 'OIHW', 'NCHW'),
                               preferred_element_type=jnp.float32)
    return y + b[None, :, None, None]


def _convT(x, w, b):
    O, I, _, _ = w.shape
    W = jnp.stack([w[:, :, (1 - di)::2, :][:, :, :, (1 - dj)::2]
                   for di in (0, 1) for dj in (0, 1)], axis=0)
    W = W.reshape(4 * O, I, 2, 2)
    p = jax.lax.conv_general_dilated(x, W, window_strides=(1, 1),
                                     padding='VALID',
                                     dimension_numbers=('NCHW', 'OIHW', 'NCHW'),
                                     preferred_element_type=jnp.float32)
    B, _, h, wd = p.shape
    p = p.reshape(B, 2, 2, O, h, wd)
    y = p.transpose(0, 3, 4, 1, 5, 2).reshape(B, O, 2 * h, 2 * wd)
    return y + b[None, :, None, None]


_ROWS = 512


def _vq_kernel(z_ref, cb_ref, quant_ref, idx_ref, loss_ref):
    z = z_ref[...]
    cb = cb_ref[...]
    zz = jnp.sum(z * z, axis=1, keepdims=True)
    cc = jnp.sum(cb * cb, axis=1)[None, :]
    cross = jax.lax.dot_general(
        z, cb, (((1,), (1,)), ((), ())),
        preferred_element_type=jnp.float32)
    d2 = zz + cc - 2.0 * cross
    idx = jnp.argmin(d2, axis=1).astype(jnp.int32)
    k = d2.shape[1]
    onehot = (idx[:, None] == jax.lax.broadcasted_iota(jnp.int32, (1, k), 1)
              ).astype(jnp.float32)
    quant = jax.lax.dot_general(
        onehot, cb, (((1,), (0,)), ((), ())),
        preferred_element_type=jnp.float32)
    quant_ref[...] = quant
    idx_ref[...] = idx.reshape(1, 1, -1)
    diff = quant - z
    part = jnp.sum(diff * diff).reshape(1, 1)
    @pl.when(pl.program_id(0) == 0)
    def _():
        loss_ref[...] = jnp.zeros((1, 1), jnp.float32)
    loss_ref[...] += part


def _vq(z_flat, codebook):
    n, d = z_flat.shape
    k = codebook.shape[0]
    nblk = n // _ROWS
    quant, idx, losssum = pl.pallas_call(
        _vq_kernel,
        grid=(nblk,),
        in_specs=[
            pl.BlockSpec((_ROWS, d), lambda i: (i, 0)),
            pl.BlockSpec((k, d), lambda i: (0, 0)),
        ],
        out_specs=[
            pl.BlockSpec((_ROWS, d), lambda i: (i, 0)),
            pl.BlockSpec((1, 1, _ROWS), lambda i: (i, 0, 0)),
            pl.BlockSpec((1, 1), lambda i: (0, 0)),
        ],
        out_shape=[
            jax.ShapeDtypeStruct((n, d), jnp.float32),
            jax.ShapeDtypeStruct((nblk, 1, _ROWS), jnp.int32),
            jax.ShapeDtypeStruct((1, 1), jnp.float32),
        ],
    )(z_flat, codebook)
    return quant, idx.reshape(n), losssum[0, 0]


_T3 = 10


def _convT3_body(xt_ref, w_ref, out_ref):
    for tau in range(110 // _T3):
        acc = jnp.zeros((_T3 * 110, 12), jnp.float32)
        for u in (0, 1):
            base = tau * (_T3 * 110) + u * 110
            lhs = jnp.concatenate(
                [xt_ref[0, base:base + _T3 * 110, :],
                 xt_ref[0, base + 1:base + 1 + _T3 * 110, :]],
                axis=1)
            acc += jax.lax.dot_general(
                lhs, w_ref[u], (((1,), (0,)), ((), ())),
                preferred_element_type=jnp.float32)
        out_ref[0, tau] = acc.T


def _convT3_pallas(y, w, b):
    B, C, R, S = y.shape
    yt = jnp.transpose(y, (0, 2, 3, 1)).astype(jnp.bfloat16)
    yt = jnp.pad(yt, ((0, 0), (0, 2), (0, 0), (0, 0)))
    yt = yt.reshape(B, 112 * 110, 128)
    wk = jnp.stack([
        jnp.concatenate([
            jnp.stack([w[c, :, 2 * u + 1 - di, 2 * v + 1 - dj]
                       for di in (0, 1) for dj in (0, 1) for c in (0, 1, 2)],
                      axis=1)
            for v in (0, 1)], axis=0)
        for u in (0, 1)], axis=0).astype(jnp.bfloat16)
    ntau = R // _T3
    p = pl.pallas_call(
        _convT3_body,
        grid=(B,),
        in_specs=[
            pl.BlockSpec((1, 112 * 110, 128), lambda bi: (bi, 0, 0)),
            pl.BlockSpec((2, 256, 12), lambda bi: (0, 0, 0)),
        ],
        out_specs=pl.BlockSpec((1, ntau, 12, _T3 * 110),
                               lambda bi: (bi, 0, 0, 0)),
        out_shape=jax.ShapeDtypeStruct((B, ntau, 12, _T3 * 110), jnp.float32),
    )(yt, wk)
    planes = p.transpose(0, 2, 1, 3).reshape(B, 12, R, S)
    xr = None
    for di in (0, 1):
        for dj in (0, 1):
            ph = planes[:, (di * 2 + dj) * 3:(di * 2 + dj) * 3 + 3]
            padded = jax.lax.pad(
                ph, jnp.float32(0),
                ((0, 0, 0), (0, 0, 0),
                 (di, 1 - di, 1), (dj, 1 - dj, 1)))
            xr = padded if xr is None else xr + padded
    xr = xr[:, :, :2 * R - 2, :2 * S - 2]
    return xr + b[None, :, None, None]


def kernel(x, enc_w1, enc_b1, enc_w2, enc_b2, enc_w3, enc_b3, codebook,
           dec_w1, dec_b1, dec_w2, dec_b2, dec_w3, dec_b3):
    beta = 0.25
    h = jax.nn.relu(_conv(x, enc_w1, enc_b1, 2, 1))
    h = jax.nn.relu(_conv(h, enc_w2, enc_b2, 2, 1))
    z = _conv(h, enc_w3, enc_b3, 1, 1)
    B, D, Hh, Ww = z.shape
    z_flat = jnp.transpose(z, (0, 2, 3, 1)).reshape(-1, D)
    quant_flat, indices, losssum = _vq(z_flat, codebook)
    quantized = jnp.transpose(quant_flat.reshape(B, Hh, Ww, D), (0, 3, 1, 2))
    codebook_loss = losssum / jnp.float32(z_flat.size)
    commitment_loss = beta * codebook_loss
    g = jax.nn.relu(_conv(quantized, dec_w1, dec_b1, 1, 1))
    g = jax.nn.relu(_convT_dilated(g, dec_w2, dec_b2))
    x_recon = _convT3_pallas(g, dec_w3, dec_b3)
    return (x_recon, codebook_loss, commitment_loss,
            indices.reshape(B, Hh, Ww))
